# baseline (speedup 1.0000x reference)
import functools

import jax
import jax.numpy as jnp
from jax import lax
from jax.experimental import pallas as pl
from jax.experimental.pallas import tpu as pltpu
from jax.experimental.pallas import tpu_sc as plsc

N = 10000
E = 320000
D = 128
ED = 32
G = 16
H1 = 32

NC = 2
NS = 16
NW = NC * NS
EPT = E // NW
K = 80
NCHUNK = EPT // K
N2 = 10240
NPS = N2 // NS
ZROWS = 128
CPS = N2 // NS


def _sc_segment(P, Q, row, col):
  mesh = plsc.VectorSubcoreMesh(core_axis_name="c", subcore_axis_name="s")

  @functools.partial(
      pl.kernel,
      mesh=mesh,
      compiler_params=pltpu.CompilerParams(use_tc_tiling_on_sc=False),
      out_type=[
          jax.ShapeDtypeStruct((NC, N2, H1), jnp.float32),
          jax.ShapeDtypeStruct((NC, N2), jnp.float32),
      ],
      scratch_types=[
          pltpu.VMEM_SHARED((N2, H1), jnp.float32),
          pltpu.VMEM_SHARED((N2,), jnp.float32),
          pltpu.VMEM((3, K), jnp.int32),
          pltpu.VMEM((3, K), jnp.int32),
          pltpu.VMEM((3, K, H1), jnp.float32),
          pltpu.VMEM((3, K, H1), jnp.float32),
          pltpu.VMEM((K,), jnp.float32),
          pltpu.VMEM((ZROWS, H1), jnp.float32),
          pltpu.VMEM((CPS,), jnp.float32),
          pltpu.SemaphoreType.DMA,
          pltpu.SemaphoreType.DMA,
          pltpu.SemaphoreType.DMA,
          pltpu.SemaphoreType.DMA,
          pltpu.SemaphoreType.DMA,
          pltpu.SemaphoreType.DMA,
          pltpu.SemaphoreType.DMA,
          pltpu.SemaphoreType.DMA,
          pltpu.SemaphoreType.DMA,
      ],
  )
  def k(p_hbm, q_hbm, row_hbm, col_hbm, s_out, c_out,
        s_sp, c_sp, colv, rowv, gv, qv, onesv, zbuf, czbuf,
        si0, si1, si2, sg0, sg1, sg2, ss0, ss1, ss2):
    c = lax.axis_index("c")
    s = lax.axis_index("s")
    wid = s * NC + c
    sem_in = (si0, si1, si2)
    sem_g = (sg0, sg1, sg2)
    sem_s = (ss0, ss1, ss2)

    zero16 = jnp.zeros((16,), jnp.float32)
    one16 = jnp.ones((16,), jnp.float32)

    def zb_body(i, _):
      zbuf[i, pl.ds(0, 16)] = zero16
      zbuf[i, pl.ds(16, 16)] = zero16
      return _
    lax.fori_loop(0, ZROWS, zb_body, None)

    def cz_body(i, _):
      czbuf[pl.ds(pl.multiple_of(i * 16, 16), 16)] = zero16
      return _
    lax.fori_loop(0, CPS // 16, cz_body, None)

    def ones_body(i, _):
      onesv[pl.ds(pl.multiple_of(i * 16, 16), 16)] = one16
      return _
    lax.fori_loop(0, K // 16, ones_body, None)

    def sz_body(j, _):
      off = pl.multiple_of(NPS * s + ZROWS * j, 8)
      pltpu.sync_copy(zbuf, s_sp.at[pl.ds(off, ZROWS), :])
      return _
    lax.fori_loop(0, NPS // ZROWS, sz_body, None)
    pltpu.sync_copy(czbuf, c_sp.at[pl.ds(pl.multiple_of(CPS * s, 8), CPS)])
    plsc.subcore_barrier()

    def ebase(g):
      return pl.multiple_of(wid * EPT + g * K, 8)

    def issue_in(g, b):
      base = ebase(g)
      pltpu.async_copy(row_hbm.at[pl.ds(base, K)], rowv.at[b], sem_in[b])
      pltpu.async_copy(col_hbm.at[pl.ds(base, K)], colv.at[b], sem_in[b])
      pltpu.async_copy(q_hbm.at[pl.ds(base, K), :], qv.at[b], sem_in[b])

    def wait_in(g, b):
      base = ebase(g)
      pltpu.make_async_copy(row_hbm.at[pl.ds(base, K)], rowv.at[b], sem_in[b]).wait()
      pltpu.make_async_copy(col_hbm.at[pl.ds(base, K)], colv.at[b], sem_in[b]).wait()
      pltpu.make_async_copy(q_hbm.at[pl.ds(base, K), :], qv.at[b], sem_in[b]).wait()

    def issue_gather(b):
      pltpu.async_copy(p_hbm.at[rowv.at[b]], gv.at[b], sem_g[b])

    def wait_gather(b):
      pltpu.make_async_copy(p_hbm.at[rowv.at[b]], gv.at[b], sem_g[b]).wait()

    def issue_scatter(b):
      pltpu.async_copy(gv.at[b], s_sp.at[colv.at[b]], sem_s[b], add=True)
      pltpu.async_copy(onesv, c_sp.at[colv.at[b]], sem_s[b], add=True)

    def wait_scatter(b):
      pltpu.make_async_copy(gv.at[b], s_sp.at[colv.at[b]], sem_s[b]).wait()
      pltpu.make_async_copy(onesv, c_sp.at[colv.at[b]], sem_s[b]).wait()

    def compute(b):
      def cmp_body(jj, _):
        r = 4 * jj
        for e in range(4):
          for h in range(2):
            seg = (gv[b, r + e, pl.ds(16 * h, 16)]
                   + qv[b, r + e, pl.ds(16 * h, 16)])
            gv[b, r + e, pl.ds(16 * h, 16)] = jnp.maximum(seg, 0.0)
        return _
      lax.fori_loop(0, K // 4, cmp_body, None)

    def step(g, j, wait_sc=True, do_gather=True, do_in=True):
      b = j % 3
      bn = (j + 1) % 3
      b2 = (j + 2) % 3
      wait_gather(b)
      if do_gather:
        wait_in(g + 1, bn)
        issue_gather(bn)
      if wait_sc:
        wait_scatter(b2)
      if do_in:
        issue_in(g + 2, b2)
      compute(b)
      issue_scatter(b)

    issue_in(0, 0)
    issue_in(1, 1)
    wait_in(0, 0)
    issue_gather(0)
    step(0, 0, wait_sc=False)
    step(1, 1)
    step(2, 2)

    def group_body(ii, _):
      g0 = 3 * ii
      step(g0, 0)
      step(g0 + 1, 1)
      step(g0 + 2, 2)
      return _
    lax.fori_loop(1, NCHUNK // 3, group_body, None)

    step(NCHUNK - 2, 0, do_in=False)
    step(NCHUNK - 1, 1, do_gather=False, do_in=False)
    wait_scatter(1)

    plsc.subcore_barrier()
    soff = pl.multiple_of(NPS * s, 8)
    pltpu.sync_copy(s_sp.at[pl.ds(soff, NPS), :], s_out.at[c, pl.ds(soff, NPS), :])
    coff = pl.multiple_of(CPS * s, 8)
    pltpu.sync_copy(c_sp.at[pl.ds(coff, CPS)], c_out.at[c, pl.ds(coff, CPS)])

  return k(P, Q, row, col)


QBLK = 8000


def _pq_body(x_ref, ea_ref, w_ref, b_ref, q_ref, p_ref):
  i = pl.program_id(0)
  q_ref[...] = (jnp.dot(ea_ref[...], w_ref[...][D:],
                        preferred_element_type=jnp.float32) + b_ref[...])

  @pl.when(i == 0)
  def _():
    p_ref[...] = jnp.dot(x_ref[...], w_ref[...][:D],
                         preferred_element_type=jnp.float32)


def _zx_body(x_ref, bt_ref, u_ref, w2a_ref, b2a_ref, o_ref):
  w2a = w2a_ref[...]
  ub = jnp.dot(u_ref[...], w2a[D + H1:], preferred_element_type=jnp.float32)
  oh = (bt_ref[...] == lax.broadcasted_iota(jnp.int32, (N, G), 1)
        ).astype(jnp.float32)
  o_ref[...] = (jnp.dot(x_ref[...], w2a[:D], preferred_element_type=jnp.float32)
                + jnp.dot(oh, ub, preferred_element_type=jnp.float32)
                + b2a_ref[...])


def _fin_body(zx_ref, s_ref, ct_ref, w1b_ref, b1b_ref, w2a_ref,
              w2b_ref, b2b_ref, o_ref):
  ssum = s_ref[0] + s_ref[1]
  ct = ct_ref[...]
  cnt = ct[:, 0:1] + ct[:, 1:2]
  mean = ssum / jnp.maximum(cnt, 1.0)
  mask = jnp.where(cnt > 0.0, 1.0, 0.0)
  agg = (jnp.dot(mean, w1b_ref[...], preferred_element_type=jnp.float32)
         + mask * b1b_ref[...])
  z1 = zx_ref[...] + jnp.dot(agg, w2a_ref[...][D:D + H1],
                             preferred_element_type=jnp.float32)
  z = jnp.maximum(z1, 0.0)
  o_ref[...] = (jnp.dot(z, w2b_ref[...], preferred_element_type=jnp.float32)
                + b2b_ref[...])


def kernel(x, edge_index, edge_attr, u, batch,
           W1a, b1a, W1b, b1b, W2a, b2a, W2b, b2b):
  row = edge_index[0]
  col = edge_index[1]

  Q, P = pl.pallas_call(
      _pq_body,
      grid=(E // QBLK,),
      in_specs=[
          pl.BlockSpec((N, D), lambda i: (0, 0)),
          pl.BlockSpec((QBLK, ED), lambda i: (i, 0)),
          pl.BlockSpec((D + ED, H1), lambda i: (0, 0)),
          pl.BlockSpec((1, H1), lambda i: (0, 0)),
      ],
      out_specs=[
          pl.BlockSpec((QBLK, H1), lambda i: (i, 0)),
          pl.BlockSpec((N, H1), lambda i: (0, 0)),
      ],
      out_shape=[
          jax.ShapeDtypeStruct((E, H1), jnp.float32),
          jax.ShapeDtypeStruct((N, H1), jnp.float32),
      ],
  )(x, edge_attr, W1a, b1a.reshape(1, H1))

  S2, C = _sc_segment(P, Q, row, col)

  bt = batch.reshape(N, 1)
  zx = pl.pallas_call(
      _zx_body,
      out_shape=jax.ShapeDtypeStruct((N, 64), jnp.float32),
  )(x, bt, u, W2a, b2a.reshape(1, 64))

  ct = C[:, :N].T
  z = pl.pallas_call(
      _fin_body,
      grid=(1,),
      in_specs=[
          pl.BlockSpec((N, 64), lambda i: (0, 0)),
          pl.BlockSpec((NC, N, H1), lambda i: (0, 0, 0)),
          pl.BlockSpec((N, NC), lambda i: (0, 0)),
          pl.BlockSpec((H1, H1), lambda i: (0, 0)),
          pl.BlockSpec((1, H1), lambda i: (0, 0)),
          pl.BlockSpec((D + H1 + D, 64), lambda i: (0, 0)),
          pl.BlockSpec((64, D), lambda i: (0, 0)),
          pl.BlockSpec((1, D), lambda i: (0, 0)),
      ],
      out_specs=pl.BlockSpec((N, D), lambda i: (0, 0)),
      out_shape=jax.ShapeDtypeStruct((N, D), jnp.float32),
  )(zx, S2, ct, W1b, b1b.reshape(1, H1), W2a, W2b, b2b.reshape(1, D))
  return z


# speedup vs baseline: 1.2804x; 1.2804x over previous
import functools

import jax
import jax.numpy as jnp
from jax import lax
from jax.experimental import pallas as pl
from jax.experimental.pallas import tpu as pltpu
from jax.experimental.pallas import tpu_sc as plsc

N = 10000
E = 320000
D = 128
ED = 32
G = 16
H1 = 32

NC = 2
NS = 16
NW = NC * NS
EPT = E // NW
K = 80
NCHUNK = EPT // K
N2 = 10240
NPS = N2 // NS
ZROWS = 128
CPS = N2 // NS


def _sc_segment(P, Q, row, col):
  mesh = plsc.VectorSubcoreMesh(core_axis_name="c", subcore_axis_name="s")

  @functools.partial(
      pl.kernel,
      mesh=mesh,
      compiler_params=pltpu.CompilerParams(use_tc_tiling_on_sc=False),
      out_type=[
          jax.ShapeDtypeStruct((NC, N2, H1), jnp.float32),
          jax.ShapeDtypeStruct((NC, N2), jnp.float32),
      ],
      scratch_types=[
          pltpu.VMEM_SHARED((N2, H1), jnp.float32),
          pltpu.VMEM_SHARED((N2,), jnp.float32),
          pltpu.VMEM((3, K), jnp.int32),
          pltpu.VMEM((3, K), jnp.int32),
          pltpu.VMEM((3, K, H1), jnp.float32),
          pltpu.VMEM((3, K // 4, 4 * H1), jnp.float32),
          pltpu.VMEM((K,), jnp.float32),
          pltpu.VMEM((ZROWS, H1), jnp.float32),
          pltpu.VMEM((CPS,), jnp.float32),
          pltpu.SemaphoreType.DMA,
          pltpu.SemaphoreType.DMA,
          pltpu.SemaphoreType.DMA,
          pltpu.SemaphoreType.DMA,
          pltpu.SemaphoreType.DMA,
          pltpu.SemaphoreType.DMA,
          pltpu.SemaphoreType.DMA,
          pltpu.SemaphoreType.DMA,
          pltpu.SemaphoreType.DMA,
      ],
  )
  def k(p_hbm, q_hbm, row_hbm, col_hbm, s_out, c_out,
        s_sp, c_sp, colv, rowv, gv, qv, onesv, zbuf, czbuf,
        si0, si1, si2, sg0, sg1, sg2, ss0, ss1, ss2):
    c = lax.axis_index("c")
    s = lax.axis_index("s")
    wid = s * NC + c
    sem_in = (si0, si1, si2)
    sem_g = (sg0, sg1, sg2)
    sem_s = (ss0, ss1, ss2)

    zero16 = jnp.zeros((16,), jnp.float32)
    one16 = jnp.ones((16,), jnp.float32)

    def zb_body(i, _):
      zbuf[i, pl.ds(0, 16)] = zero16
      zbuf[i, pl.ds(16, 16)] = zero16
      return _
    lax.fori_loop(0, ZROWS, zb_body, None)

    def cz_body(i, _):
      czbuf[pl.ds(pl.multiple_of(i * 16, 16), 16)] = zero16
      return _
    lax.fori_loop(0, CPS // 16, cz_body, None)

    def ones_body(i, _):
      onesv[pl.ds(pl.multiple_of(i * 16, 16), 16)] = one16
      return _
    lax.fori_loop(0, K // 16, ones_body, None)

    def sz_body(j, _):
      off = pl.multiple_of(NPS * s + ZROWS * j, 8)
      pltpu.sync_copy(zbuf, s_sp.at[pl.ds(off, ZROWS), :])
      return _
    lax.fori_loop(0, NPS // ZROWS, sz_body, None)
    pltpu.sync_copy(czbuf, c_sp.at[pl.ds(pl.multiple_of(CPS * s, 8), CPS)])
    plsc.subcore_barrier()

    def ebase(g):
      return pl.multiple_of(wid * EPT + g * K, 8)

    def qbase(g):
      return pl.multiple_of((wid * EPT) // 4 + g * (K // 4), 4)

    def issue_in(g, b):
      base = ebase(g)
      pltpu.async_copy(row_hbm.at[pl.ds(base, K)], rowv.at[b], sem_in[b])
      pltpu.async_copy(col_hbm.at[pl.ds(base, K)], colv.at[b], sem_in[b])
      pltpu.async_copy(q_hbm.at[pl.ds(qbase(g), K // 4), :], qv.at[b], sem_in[b])

    def wait_in(g, b):
      base = ebase(g)
      pltpu.make_async_copy(row_hbm.at[pl.ds(base, K)], rowv.at[b], sem_in[b]).wait()
      pltpu.make_async_copy(col_hbm.at[pl.ds(base, K)], colv.at[b], sem_in[b]).wait()
      pltpu.make_async_copy(q_hbm.at[pl.ds(qbase(g), K // 4), :], qv.at[b], sem_in[b]).wait()

    def issue_gather(b):
      pltpu.async_copy(p_hbm.at[rowv.at[b]], gv.at[b], sem_g[b])

    def wait_gather(b):
      pltpu.make_async_copy(p_hbm.at[rowv.at[b]], gv.at[b], sem_g[b]).wait()

    def issue_scatter(b):
      pltpu.async_copy(gv.at[b], s_sp.at[colv.at[b]], sem_s[b], add=True)
      pltpu.async_copy(onesv, c_sp.at[colv.at[b]], sem_s[b], add=True)

    def wait_scatter(b):
      pltpu.make_async_copy(gv.at[b], s_sp.at[colv.at[b]], sem_s[b]).wait()
      pltpu.make_async_copy(onesv, c_sp.at[colv.at[b]], sem_s[b]).wait()

    def compute(b):
      def cmp_body(jj, _):
        r = 4 * jj
        for e in range(4):
          for h in range(2):
            seg = (gv[b, r + e, pl.ds(16 * h, 16)]
                   + qv[b, jj, pl.ds(32 * e + 16 * h, 16)])
            gv[b, r + e, pl.ds(16 * h, 16)] = jnp.maximum(seg, 0.0)
        return _
      lax.fori_loop(0, K // 4, cmp_body, None)

    def step(g, j, wait_sc=True, do_gather=True, do_in=True):
      b = j % 3
      bn = (j + 1) % 3
      b2 = (j + 2) % 3
      wait_gather(b)
      if do_gather:
        wait_in(g + 1, bn)
        issue_gather(bn)
      if wait_sc:
        wait_scatter(b2)
      if do_in:
        issue_in(g + 2, b2)
      compute(b)
      issue_scatter(b)

    issue_in(0, 0)
    issue_in(1, 1)
    wait_in(0, 0)
    issue_gather(0)
    step(0, 0, wait_sc=False)
    step(1, 1)
    step(2, 2)

    def group_body(ii, _):
      g0 = 3 * ii
      step(g0, 0)
      step(g0 + 1, 1)
      step(g0 + 2, 2)
      return _
    lax.fori_loop(1, NCHUNK // 3, group_body, None)

    step(NCHUNK - 2, 0, do_in=False)
    step(NCHUNK - 1, 1, do_gather=False, do_in=False)
    wait_scatter(1)

    plsc.subcore_barrier()
    soff = pl.multiple_of(NPS * s, 8)
    pltpu.sync_copy(s_sp.at[pl.ds(soff, NPS), :], s_out.at[c, pl.ds(soff, NPS), :])
    coff = pl.multiple_of(CPS * s, 8)
    pltpu.sync_copy(c_sp.at[pl.ds(coff, CPS)], c_out.at[c, pl.ds(coff, CPS)])

  return k(P, Q, row, col)


E4 = E // 4
QBLK4 = 2000


def _pq_body(x_ref, ea_ref, w_ref, b_ref, q_ref, p_ref):
  i = pl.program_id(0)
  w = w_ref[...][D:]
  ea = ea_ref[...]
  for j in range(4):
    q_ref[:, 32 * j:32 * (j + 1)] = (
        jnp.dot(ea[:, 32 * j:32 * (j + 1)], w,
                preferred_element_type=jnp.float32) + b_ref[...])

  @pl.when(i == 0)
  def _():
    p_ref[...] = jnp.dot(x_ref[...], w_ref[...][:D],
                         preferred_element_type=jnp.float32)


def _zx_body(x_ref, bt_ref, u_ref, w2a_ref, b2a_ref, o_ref):
  w2a = w2a_ref[...]
  ub = jnp.dot(u_ref[...], w2a[D + H1:], preferred_element_type=jnp.float32)
  oh = (bt_ref[...] == lax.broadcasted_iota(jnp.int32, (N, G), 1)
        ).astype(jnp.float32)
  o_ref[...] = (jnp.dot(x_ref[...], w2a[:D], preferred_element_type=jnp.float32)
                + jnp.dot(oh, ub, preferred_element_type=jnp.float32)
                + b2a_ref[...])


def _fin_body(zx_ref, s_ref, ct_ref, w1b_ref, b1b_ref, w2a_ref,
              w2b_ref, b2b_ref, o_ref):
  ssum = s_ref[0] + s_ref[1]
  ct = ct_ref[...]
  cnt = ct[:, 0:1] + ct[:, 1:2]
  mean = ssum / jnp.maximum(cnt, 1.0)
  mask = jnp.where(cnt > 0.0, 1.0, 0.0)
  agg = (jnp.dot(mean, w1b_ref[...], preferred_element_type=jnp.float32)
         + mask * b1b_ref[...])
  z1 = zx_ref[...] + jnp.dot(agg, w2a_ref[...][D:D + H1],
                             preferred_element_type=jnp.float32)
  z = jnp.maximum(z1, 0.0)
  o_ref[...] = (jnp.dot(z, w2b_ref[...], preferred_element_type=jnp.float32)
                + b2b_ref[...])


def kernel(x, edge_index, edge_attr, u, batch,
           W1a, b1a, W1b, b1b, W2a, b2a, W2b, b2b):
  row = edge_index[0]
  col = edge_index[1]

  ea4 = edge_attr.reshape(E4, 4 * ED)
  Q, P = pl.pallas_call(
      _pq_body,
      grid=(E4 // QBLK4,),
      in_specs=[
          pl.BlockSpec((N, D), lambda i: (0, 0)),
          pl.BlockSpec((QBLK4, 4 * ED), lambda i: (i, 0)),
          pl.BlockSpec((D + ED, H1), lambda i: (0, 0)),
          pl.BlockSpec((1, H1), lambda i: (0, 0)),
      ],
      out_specs=[
          pl.BlockSpec((QBLK4, 4 * H1), lambda i: (i, 0)),
          pl.BlockSpec((N, H1), lambda i: (0, 0)),
      ],
      out_shape=[
          jax.ShapeDtypeStruct((E4, 4 * H1), jnp.float32),
          jax.ShapeDtypeStruct((N, H1), jnp.float32),
      ],
  )(x, ea4, W1a, b1a.reshape(1, H1))

  S2, C = _sc_segment(P, Q, row, col)

  bt = batch.reshape(N, 1)
  zx = pl.pallas_call(
      _zx_body,
      out_shape=jax.ShapeDtypeStruct((N, 64), jnp.float32),
  )(x, bt, u, W2a, b2a.reshape(1, 64))

  ct = C[:, :N].T
  z = pl.pallas_call(
      _fin_body,
      grid=(1,),
      in_specs=[
          pl.BlockSpec((N, 64), lambda i: (0, 0)),
          pl.BlockSpec((NC, N, H1), lambda i: (0, 0, 0)),
          pl.BlockSpec((N, NC), lambda i: (0, 0)),
          pl.BlockSpec((H1, H1), lambda i: (0, 0)),
          pl.BlockSpec((1, H1), lambda i: (0, 0)),
          pl.BlockSpec((D + H1 + D, 64), lambda i: (0, 0)),
          pl.BlockSpec((64, D), lambda i: (0, 0)),
          pl.BlockSpec((1, D), lambda i: (0, 0)),
      ],
      out_specs=pl.BlockSpec((N, D), lambda i: (0, 0)),
      out_shape=jax.ShapeDtypeStruct((N, D), jnp.float32),
  )(zx, S2, ct, W1b, b1b.reshape(1, H1), W2a, W2b, b2b.reshape(1, D))
  return z
